# baseline (speedup 1.0000x reference)
import functools

import jax
import jax.numpy as jnp
from jax.experimental import pallas as pl
from jax.experimental.pallas import tpu as pltpu

_C = 256
_NHEAD = 75
_PAD = 128
_GS = [128, 64, 32, 16, 8]
_SQ2 = 1.4142135623730951
_ISQ2 = 0.7071067811865476


def _dot(a, b):
    return jax.lax.dot_general(a, b, (((1,), (0,)), ((), ())),
                               preferred_element_type=jnp.float32)


def _anchors(a4_ref):
    field = jax.lax.broadcasted_iota(jnp.int32, (4, 1), 0)
    off = 0
    for g in _GS:
        na = 15 * g * g
        lg = g.bit_length() - 1
        m = jax.lax.broadcasted_iota(jnp.int32, (1, na), 1)
        k = m >> (2 * lg)
        p2 = m & (g * g - 1)
        i = p2 >> lg
        j = p2 & (g - 1)
        stride = 512.0 / g
        xv = i.astype(jnp.float32) * stride
        yv = j.astype(jnp.float32) * stride
        ks = (k * 171) >> 9
        km = k - 3 * ks
        s = (32 << ks).astype(jnp.float32)
        wv = s * jnp.where(km == 0, _ISQ2, jnp.where(km == 1, 1.0, _SQ2))
        hv = s * jnp.where(km == 0, _SQ2, jnp.where(km == 1, 1.0, _ISQ2))
        a4_ref[:, off:off + na] = jnp.where(
            field == 0, xv,
            jnp.where(field == 1, yv, jnp.where(field == 2, wv, hv)))
        off += na


def _finish(acc, cb_ref, wh_ref, hb_ref, cls_ref, box_ref):
    t = jnp.maximum(acc + cb_ref[:, 0:1], 0.0)
    res = _dot(wh_ref[...], t.astype(jnp.bfloat16)) + hb_ref[:, 0:1]
    rows = cls_ref.shape[-2]
    w = cls_ref.shape[-1]
    cls_ref[...] = res[:15].reshape(1, 15, rows, w)
    box_ref[...] = res[15:].reshape(1, 60, rows, w)


def _body_l0(xp_ref, w9_ref, cb_ref, wh_ref, hb_ref, cls_ref, box_ref, *,
             g, tn):
    p = pl.program_id(0)
    base = _PAD + p * tn
    wins = [xp_ref[:, pl.ds(base + (dy - 1) * g, tn)] for dy in range(3)]
    lane = jax.lax.broadcasted_iota(jnp.int32, (1, tn), 1)
    j = lane % g
    acc = None
    for dx in range(3):
        s = _dot(w9_ref[dx], wins[0])
        for dy in (1, 2):
            s += _dot(w9_ref[dy * 3 + dx], wins[dy])
        if dx == 0:
            s = jnp.where(j != 0, pltpu.roll(s, 1, axis=1), 0.0)
        elif dx == 2:
            s = jnp.where(j != g - 1, pltpu.roll(s, tn - 1, axis=1), 0.0)
        acc = s if acc is None else acc + s
    _finish(acc, cb_ref, wh_ref, hb_ref, cls_ref, box_ref)


def _tail_level(x_ref, w9_ref, cb_ref, wh_ref, hb_ref, cls_ref, box_ref, g):
    tn = g * g
    win = x_ref[...].astype(jnp.bfloat16)
    n = jax.lax.broadcasted_iota(jnp.int32, (1, tn), 1)
    wins = [
        jnp.where(n >= g, pltpu.roll(win, g, axis=1), jnp.bfloat16(0.0)),
        win,
        jnp.where(n < tn - g, pltpu.roll(win, tn - g, axis=1),
                  jnp.bfloat16(0.0)),
    ]
    j = n % g
    acc = None
    for dx in range(3):
        s = _dot(w9_ref[dx], wins[0])
        for dy in (1, 2):
            s += _dot(w9_ref[dy * 3 + dx], wins[dy])
        if dx == 0:
            s = jnp.where(j != 0, pltpu.roll(s, 1, axis=1), 0.0)
        elif dx == 2:
            s = jnp.where(j != g - 1, pltpu.roll(s, tn - 1, axis=1), 0.0)
        acc = s if acc is None else acc + s
    _finish(acc, cb_ref, wh_ref, hb_ref, cls_ref, box_ref)


def _body_tails(x1, x2, x3, x4, w9_ref, cb_ref, wh_ref, hb_ref, *out_refs):
    for idx, x_ref in enumerate((x1, x2, x3, x4)):
        _tail_level(x_ref, w9_ref, cb_ref, wh_ref, hb_ref,
                    out_refs[idx], out_refs[4 + idx], _GS[1 + idx])


def _full(shape):
    return pl.BlockSpec(shape, lambda p: tuple(0 for _ in shape))


def kernel(images, feat0, feat1, feat2, feat3, feat4,
           conv_w, conv_b, cls_w, cls_b, bbox_w, bbox_b):
    w9 = conv_w.transpose(2, 3, 0, 1).reshape(9, _C, _C).astype(jnp.bfloat16)
    cb = conv_b.reshape(_C, 1)
    wh = jnp.concatenate([cls_w, bbox_w], axis=0).astype(jnp.bfloat16)
    hb = jnp.concatenate([cls_b, bbox_b]).reshape(_NHEAD, 1)

    g0, n0, tn = 128, 128 * 128, 4096
    grid = n0 // tn
    rpt = tn // g0
    xp = jnp.pad(feat0.reshape(_C, n0).astype(jnp.bfloat16),
                 ((0, 0), (_PAD, _PAD)))
    cls0, box0 = pl.pallas_call(
        functools.partial(_body_l0, g=g0, tn=tn),
        grid=(grid,),
        in_specs=[_full(xp.shape), _full(w9.shape), _full(cb.shape),
                  _full(wh.shape), _full(hb.shape)],
        out_specs=[
            pl.BlockSpec((1, 15, rpt, g0), lambda p: (0, 0, p, 0)),
            pl.BlockSpec((1, 60, rpt, g0), lambda p: (0, 0, p, 0)),
        ],
        out_shape=[
            jax.ShapeDtypeStruct((1, 15, g0, g0), jnp.float32),
            jax.ShapeDtypeStruct((1, 60, g0, g0), jnp.float32),
        ],
    )(xp, w9, cb, wh, hb)

    gs = _GS[1:]
    feats = [feat1.reshape(_C, 64 * 64), feat2.reshape(_C, 32 * 32),
             feat3.reshape(_C, 16 * 16), feat4.reshape(_C, 8 * 8)]
    out_shapes = ([jax.ShapeDtypeStruct((1, 15, g, g), jnp.float32)
                   for g in gs]
                  + [jax.ShapeDtypeStruct((1, 60, g, g), jnp.float32)
                     for g in gs])
    outs = pl.pallas_call(
        _body_tails,
        grid=(1,),
        in_specs=[_full(f.shape) for f in feats]
        + [_full(w9.shape), _full(cb.shape), _full(wh.shape), _full(hb.shape)],
        out_specs=[_full(s.shape) for s in out_shapes],
        out_shape=out_shapes,
    )(*feats, w9, cb, wh, hb)

    total_anchors = sum(15 * g * g for g in _GS)
    a4 = pl.pallas_call(
        _anchors,
        out_shape=jax.ShapeDtypeStruct((4, total_anchors), jnp.float32),
    )()
    anchors = a4.transpose(1, 0)
    return (anchors, cls0, *outs[:4], box0, *outs[4:8])


# speedup vs baseline: 1.0233x; 1.0233x over previous
import functools

import jax
import jax.numpy as jnp
from jax.experimental import pallas as pl
from jax.experimental.pallas import tpu as pltpu

_C = 256
_NHEAD = 75
_PAD = 128
_GS = [128, 64, 32, 16, 8]
_SQ2 = 1.4142135623730951
_ISQ2 = 0.7071067811865476


def _dot(a, b):
    return jax.lax.dot_general(a, b, (((1,), (0,)), ((), ())),
                               preferred_element_type=jnp.float32)


def _anchors(a4_ref):
    field = jax.lax.broadcasted_iota(jnp.int32, (4, 1), 0)
    off = 0
    for g in _GS:
        na = 15 * g * g
        lg = g.bit_length() - 1
        m = jax.lax.broadcasted_iota(jnp.int32, (1, na), 1)
        k = m >> (2 * lg)
        p2 = m & (g * g - 1)
        i = p2 >> lg
        j = p2 & (g - 1)
        stride = 512.0 / g
        xv = i.astype(jnp.float32) * stride
        yv = j.astype(jnp.float32) * stride
        ks = (k * 171) >> 9
        km = k - 3 * ks
        s = (32 << ks).astype(jnp.float32)
        wv = s * jnp.where(km == 0, _ISQ2, jnp.where(km == 1, 1.0, _SQ2))
        hv = s * jnp.where(km == 0, _SQ2, jnp.where(km == 1, 1.0, _ISQ2))
        a4_ref[:, off:off + na] = jnp.where(
            field == 0, xv,
            jnp.where(field == 1, yv, jnp.where(field == 2, wv, hv)))
        off += na


def _finish(acc, cb_ref, wh_ref, hb_ref, cls_ref, box_ref):
    t = jnp.maximum(acc + cb_ref[:, 0:1], 0.0)
    res = _dot(wh_ref[...], t.astype(jnp.bfloat16)) + hb_ref[:, 0:1]
    rows = cls_ref.shape[-2]
    w = cls_ref.shape[-1]
    cls_ref[...] = res[:15].reshape(1, 15, rows, w)
    box_ref[...] = res[15:].reshape(1, 60, rows, w)


def _body_l0(x_ref, w9_ref, cb_ref, wh_ref, hb_ref, cls_ref, box_ref, *,
             g, tn):
    n0 = x_ref.shape[-1]
    rb = tn // 128
    last = n0 // tn - 1
    p = pl.program_id(0)
    lane = jax.lax.broadcasted_iota(jnp.int32, (1, tn), 1)
    n = lane + p * tn
    off_m = 128 * jnp.maximum(p * rb - 1, 0)
    off_p = 128 * jnp.minimum(p * rb + 1, last * rb)
    win_m = pltpu.roll(x_ref[:, pl.ds(off_m, tn)].astype(jnp.bfloat16),
                       jnp.where(p == 0, g, 0), axis=1)
    win_m = jnp.where((n >= g) | (p != 0), win_m, jnp.bfloat16(0.0))
    win_p = pltpu.roll(x_ref[:, pl.ds(off_p, tn)].astype(jnp.bfloat16),
                       jnp.where(p == last, tn - g, 0), axis=1)
    win_p = jnp.where((n < n0 - g) | (p != last), win_p, jnp.bfloat16(0.0))
    wins = [win_m,
            x_ref[:, pl.ds(128 * (p * rb), tn)].astype(jnp.bfloat16),
            win_p]
    j = lane % g
    acc = None
    for dx in range(3):
        s = _dot(w9_ref[dx], wins[0])
        for dy in (1, 2):
            s += _dot(w9_ref[dy * 3 + dx], wins[dy])
        if dx == 0:
            s = jnp.where(j != 0, pltpu.roll(s, 1, axis=1), 0.0)
        elif dx == 2:
            s = jnp.where(j != g - 1, pltpu.roll(s, tn - 1, axis=1), 0.0)
        acc = s if acc is None else acc + s
    _finish(acc, cb_ref, wh_ref, hb_ref, cls_ref, box_ref)


def _tail_level(x_ref, w9_ref, cb_ref, wh_ref, hb_ref, cls_ref, box_ref, g):
    tn = g * g
    win = x_ref[...].astype(jnp.bfloat16)
    n = jax.lax.broadcasted_iota(jnp.int32, (1, tn), 1)
    wins = [
        jnp.where(n >= g, pltpu.roll(win, g, axis=1), jnp.bfloat16(0.0)),
        win,
        jnp.where(n < tn - g, pltpu.roll(win, tn - g, axis=1),
                  jnp.bfloat16(0.0)),
    ]
    j = n % g
    acc = None
    for dx in range(3):
        s = _dot(w9_ref[dx], wins[0])
        for dy in (1, 2):
            s += _dot(w9_ref[dy * 3 + dx], wins[dy])
        if dx == 0:
            s = jnp.where(j != 0, pltpu.roll(s, 1, axis=1), 0.0)
        elif dx == 2:
            s = jnp.where(j != g - 1, pltpu.roll(s, tn - 1, axis=1), 0.0)
        acc = s if acc is None else acc + s
    _finish(acc, cb_ref, wh_ref, hb_ref, cls_ref, box_ref)


def _body_tails(x1, x2, x3, x4, w9_ref, cb_ref, wh_ref, hb_ref, *out_refs):
    for idx, x_ref in enumerate((x1, x2, x3, x4)):
        _tail_level(x_ref, w9_ref, cb_ref, wh_ref, hb_ref,
                    out_refs[idx], out_refs[4 + idx], _GS[1 + idx])


def _full(shape):
    return pl.BlockSpec(shape, lambda p: tuple(0 for _ in shape))


def kernel(images, feat0, feat1, feat2, feat3, feat4,
           conv_w, conv_b, cls_w, cls_b, bbox_w, bbox_b):
    w9 = conv_w.transpose(2, 3, 0, 1).reshape(9, _C, _C).astype(jnp.bfloat16)
    cb = conv_b.reshape(_C, 1)
    wh = jnp.concatenate([cls_w, bbox_w], axis=0).astype(jnp.bfloat16)
    hb = jnp.concatenate([cls_b, bbox_b]).reshape(_NHEAD, 1)

    g0, n0, tn = 128, 128 * 128, 2048
    grid = n0 // tn
    rpt = tn // g0
    x0 = feat0.reshape(_C, n0)
    cls0, box0 = pl.pallas_call(
        functools.partial(_body_l0, g=g0, tn=tn),
        grid=(grid,),
        in_specs=[_full(x0.shape), _full(w9.shape), _full(cb.shape),
                  _full(wh.shape), _full(hb.shape)],
        out_specs=[
            pl.BlockSpec((1, 15, rpt, g0), lambda p: (0, 0, p, 0)),
            pl.BlockSpec((1, 60, rpt, g0), lambda p: (0, 0, p, 0)),
        ],
        out_shape=[
            jax.ShapeDtypeStruct((1, 15, g0, g0), jnp.float32),
            jax.ShapeDtypeStruct((1, 60, g0, g0), jnp.float32),
        ],
    )(x0, w9, cb, wh, hb)

    gs = _GS[1:]
    feats = [feat1.reshape(_C, 64 * 64), feat2.reshape(_C, 32 * 32),
             feat3.reshape(_C, 16 * 16), feat4.reshape(_C, 8 * 8)]
    out_shapes = ([jax.ShapeDtypeStruct((1, 15, g, g), jnp.float32)
                   for g in gs]
                  + [jax.ShapeDtypeStruct((1, 60, g, g), jnp.float32)
                     for g in gs])
    outs = pl.pallas_call(
        _body_tails,
        grid=(1,),
        in_specs=[_full(f.shape) for f in feats]
        + [_full(w9.shape), _full(cb.shape), _full(wh.shape), _full(hb.shape)],
        out_specs=[_full(s.shape) for s in out_shapes],
        out_shape=out_shapes,
    )(*feats, w9, cb, wh, hb)

    total_anchors = sum(15 * g * g for g in _GS)
    a4 = pl.pallas_call(
        _anchors,
        out_shape=jax.ShapeDtypeStruct((4, total_anchors), jnp.float32),
    )()
    anchors = a4.transpose(1, 0)
    return (anchors, cls0, *outs[:4], box0, *outs[4:8])


# speedup vs baseline: 1.0255x; 1.0021x over previous
import functools

import jax
import jax.numpy as jnp
from jax.experimental import pallas as pl
from jax.experimental.pallas import tpu as pltpu

_C = 256
_NHEAD = 75
_PAD = 128
_GS = [128, 64, 32, 16, 8]
_SQ2 = 1.4142135623730951
_ISQ2 = 0.7071067811865476


def _dot(a, b):
    return jax.lax.dot_general(a, b, (((1,), (0,)), ((), ())),
                               preferred_element_type=jnp.float32)


def _anchors(a4_ref):
    field = jax.lax.broadcasted_iota(jnp.int32, (4, 1), 0)
    off = 0
    for g in _GS:
        na = 15 * g * g
        lg = g.bit_length() - 1
        m = jax.lax.broadcasted_iota(jnp.int32, (1, na), 1)
        k = m >> (2 * lg)
        p2 = m & (g * g - 1)
        i = p2 >> lg
        j = p2 & (g - 1)
        stride = 512.0 / g
        xv = i.astype(jnp.float32) * stride
        yv = j.astype(jnp.float32) * stride
        ks = (k * 171) >> 9
        km = k - 3 * ks
        s = (32 << ks).astype(jnp.float32)
        wv = s * jnp.where(km == 0, _ISQ2, jnp.where(km == 1, 1.0, _SQ2))
        hv = s * jnp.where(km == 0, _SQ2, jnp.where(km == 1, 1.0, _ISQ2))
        a4_ref[:, off:off + na] = jnp.where(
            field == 0, xv,
            jnp.where(field == 1, yv, jnp.where(field == 2, wv, hv)))
        off += na


def _finish(acc, cb_ref, wh_ref, hb_ref, cls_ref, box_ref):
    t = jnp.maximum(acc + cb_ref[:, 0:1], 0.0)
    res = _dot(wh_ref[...], t.astype(jnp.bfloat16)) + hb_ref[:, 0:1]
    rows = cls_ref.shape[-2]
    w = cls_ref.shape[-1]
    cls_ref[...] = res[:15].reshape(1, 15, rows, w)
    box_ref[...] = res[15:].reshape(1, 60, rows, w)


def _body_l0(x_ref, w9_ref, cb_ref, wh_ref, hb_ref, cls_ref, box_ref, a4_ref,
             *, g, tn):
    n0 = x_ref.shape[-1]
    rb = tn // 128
    last = n0 // tn - 1
    p = pl.program_id(0)
    lane = jax.lax.broadcasted_iota(jnp.int32, (1, tn), 1)
    n = lane + p * tn
    off_m = 128 * jnp.maximum(p * rb - 1, 0)
    off_p = 128 * jnp.minimum(p * rb + 1, last * rb)
    win_m = pltpu.roll(x_ref[:, pl.ds(off_m, tn)].astype(jnp.bfloat16),
                       jnp.where(p == 0, g, 0), axis=1)
    win_m = jnp.where((n >= g) | (p != 0), win_m, jnp.bfloat16(0.0))
    win_p = pltpu.roll(x_ref[:, pl.ds(off_p, tn)].astype(jnp.bfloat16),
                       jnp.where(p == last, tn - g, 0), axis=1)
    win_p = jnp.where((n < n0 - g) | (p != last), win_p, jnp.bfloat16(0.0))
    wins = [win_m,
            x_ref[:, pl.ds(128 * (p * rb), tn)].astype(jnp.bfloat16),
            win_p]
    j = lane % g
    acc = None
    for dx in range(3):
        s = _dot(w9_ref[dx], wins[0])
        for dy in (1, 2):
            s += _dot(w9_ref[dy * 3 + dx], wins[dy])
        if dx == 0:
            s = jnp.where(j != 0, pltpu.roll(s, 1, axis=1), 0.0)
        elif dx == 2:
            s = jnp.where(j != g - 1, pltpu.roll(s, tn - 1, axis=1), 0.0)
        acc = s if acc is None else acc + s
    _finish(acc, cb_ref, wh_ref, hb_ref, cls_ref, box_ref)

    @pl.when(p == 0)
    def _():
        _anchors(a4_ref)


def _tail_level(x_ref, w9_ref, cb_ref, wh_ref, hb_ref, cls_ref, box_ref, g):
    tn = g * g
    win = x_ref[...].astype(jnp.bfloat16)
    n = jax.lax.broadcasted_iota(jnp.int32, (1, tn), 1)
    wins = [
        jnp.where(n >= g, pltpu.roll(win, g, axis=1), jnp.bfloat16(0.0)),
        win,
        jnp.where(n < tn - g, pltpu.roll(win, tn - g, axis=1),
                  jnp.bfloat16(0.0)),
    ]
    j = n % g
    acc = None
    for dx in range(3):
        s = _dot(w9_ref[dx], wins[0])
        for dy in (1, 2):
            s += _dot(w9_ref[dy * 3 + dx], wins[dy])
        if dx == 0:
            s = jnp.where(j != 0, pltpu.roll(s, 1, axis=1), 0.0)
        elif dx == 2:
            s = jnp.where(j != g - 1, pltpu.roll(s, tn - 1, axis=1), 0.0)
        acc = s if acc is None else acc + s
    _finish(acc, cb_ref, wh_ref, hb_ref, cls_ref, box_ref)


def _body_tails(x1, x2, x3, x4, w9_ref, cb_ref, wh_ref, hb_ref, *out_refs):
    for idx, x_ref in enumerate((x1, x2, x3, x4)):
        _tail_level(x_ref, w9_ref, cb_ref, wh_ref, hb_ref,
                    out_refs[idx], out_refs[4 + idx], _GS[1 + idx])


def _full(shape):
    return pl.BlockSpec(shape, lambda p: tuple(0 for _ in shape))


def kernel(images, feat0, feat1, feat2, feat3, feat4,
           conv_w, conv_b, cls_w, cls_b, bbox_w, bbox_b):
    w9 = conv_w.transpose(2, 3, 0, 1).reshape(9, _C, _C).astype(jnp.bfloat16)
    cb = conv_b.reshape(_C, 1)
    wh = jnp.concatenate([cls_w, bbox_w], axis=0).astype(jnp.bfloat16)
    hb = jnp.concatenate([cls_b, bbox_b]).reshape(_NHEAD, 1)

    g0, n0, tn = 128, 128 * 128, 2048
    grid = n0 // tn
    rpt = tn // g0
    total_anchors = sum(15 * g * g for g in _GS)
    x0 = feat0.reshape(_C, n0)
    cls0, box0, a4 = pl.pallas_call(
        functools.partial(_body_l0, g=g0, tn=tn),
        grid=(grid,),
        in_specs=[_full(x0.shape), _full(w9.shape), _full(cb.shape),
                  _full(wh.shape), _full(hb.shape)],
        out_specs=[
            pl.BlockSpec((1, 15, rpt, g0), lambda p: (0, 0, p, 0)),
            pl.BlockSpec((1, 60, rpt, g0), lambda p: (0, 0, p, 0)),
            pl.BlockSpec((4, total_anchors), lambda p: (0, 0)),
        ],
        out_shape=[
            jax.ShapeDtypeStruct((1, 15, g0, g0), jnp.float32),
            jax.ShapeDtypeStruct((1, 60, g0, g0), jnp.float32),
            jax.ShapeDtypeStruct((4, total_anchors), jnp.float32),
        ],
    )(x0, w9, cb, wh, hb)

    gs = _GS[1:]
    feats = [feat1.reshape(_C, 64 * 64), feat2.reshape(_C, 32 * 32),
             feat3.reshape(_C, 16 * 16), feat4.reshape(_C, 8 * 8)]
    out_shapes = ([jax.ShapeDtypeStruct((1, 15, g, g), jnp.float32)
                   for g in gs]
                  + [jax.ShapeDtypeStruct((1, 60, g, g), jnp.float32)
                     for g in gs])
    outs = pl.pallas_call(
        _body_tails,
        grid=(1,),
        in_specs=[_full(f.shape) for f in feats]
        + [_full(w9.shape), _full(cb.shape), _full(wh.shape), _full(hb.shape)],
        out_specs=[_full(s.shape) for s in out_shapes],
        out_shape=out_shapes,
    )(*feats, w9, cb, wh, hb)

    anchors = a4.transpose(1, 0)
    return (anchors, cls0, *outs[:4], box0, *outs[4:8])
